# baseline (speedup 1.0000x reference)
import functools

import jax
import jax.numpy as jnp
from jax import lax
from jax.experimental import pallas as pl
from jax.experimental.pallas import tpu as pltpu
from jax.experimental.pallas import tpu_sc as plsc

N = 10000
NP = 10240
E = 160000
D_IN = 256
F = 16
NCLS = 7

NC = 2
NS = 16
L = 16
NW = NC * NS
KPW = 40
EP = NW * KPW * 128
RPT = NP // NS
BM = 1024


def _sc_mesh():
    return plsc.VectorSubcoreMesh(core_axis_name="c", subcore_axis_name="s")


_SC_PARAMS = pltpu.CompilerParams(needs_layout_passes=False,
                                  use_tc_tiling_on_sc=False)


def _degree(dst3):
    @functools.partial(
        pl.kernel,
        mesh=_sc_mesh(),
        compiler_params=_SC_PARAMS,
        out_type=jax.ShapeDtypeStruct((NC, NP), jnp.float32),
        scratch_types=[
            pltpu.VMEM((KPW, 128), jnp.int32),
            pltpu.VMEM((NP,), jnp.float32),
            pltpu.VMEM_SHARED((NS, NP), jnp.float32),
            pltpu.VMEM((NS, RPT), jnp.float32),
            pltpu.VMEM((RPT,), jnp.float32),
        ],
    )
    def k(dst_hbm, out_hbm, dst_v, deg_v, deg_sh, red_v, sum_v):
        c = lax.axis_index("c")
        s = lax.axis_index("s")
        wid = s * NC + c
        pltpu.sync_copy(dst_hbm.at[wid], dst_v)

        z16 = jnp.zeros((L,), jnp.float32)

        def zero_body(i, carry):
            deg_v[pl.ds(i * L, L)] = z16
            return carry

        lax.fori_loop(0, NP // L, zero_body, None)

        ones16 = jnp.ones((L,), jnp.float32)

        def acc_body(j, carry):
            for l in range(128 // L):
                idx = dst_v[j, pl.ds(l * L, L)]
                plsc.addupdate_scatter(deg_v, [idx], ones16)
            return carry

        lax.fori_loop(0, KPW, acc_body, None)

        pltpu.sync_copy(deg_v, deg_sh.at[s])
        plsc.subcore_barrier()

        for r in range(NS):
            pltpu.sync_copy(deg_sh.at[r, pl.ds(s * RPT, RPT)], red_v.at[r])

        def red_body(t, carry):
            acc = red_v[0, pl.ds(t * L, L)]
            for r in range(1, NS):
                acc = acc + red_v[r, pl.ds(t * L, L)]
            sum_v[pl.ds(t * L, L)] = acc
            return carry

        lax.fori_loop(0, RPT // L, red_body, None)
        pltpu.sync_copy(sum_v, out_hbm.at[c, pl.ds(s * RPT, RPT)])

    return k(dst3)


KE = E // NC // 128
CH = 25


def _aggregate(uT, src2, dst2):
    @functools.partial(
        pl.kernel,
        mesh=_sc_mesh(),
        compiler_params=_SC_PARAMS,
        out_type=jax.ShapeDtypeStruct((NC, F, NP), jnp.float32),
        scratch_types=[
            pltpu.VMEM((CH, 128), jnp.int32),
            pltpu.VMEM((CH, 128), jnp.int32),
            pltpu.VMEM((NP,), jnp.float32),
            pltpu.VMEM((NP,), jnp.float32),
        ],
    )
    def k(u_hbm, src_hbm, dst_hbm, out_hbm, src_v, dst_v, u_v, acc_v):
        c = lax.axis_index("c")
        s = lax.axis_index("s")
        pltpu.sync_copy(u_hbm.at[s], u_v)

        z16 = jnp.zeros((L,), jnp.float32)

        def zero_body(i, carry):
            acc_v[pl.ds(i * L, L)] = z16
            return carry

        lax.fori_loop(0, NP // L, zero_body, None)

        def chunk_body(t, carry):
            pltpu.sync_copy(src_hbm.at[c, pl.ds(t * CH, CH)], src_v)
            pltpu.sync_copy(dst_hbm.at[c, pl.ds(t * CH, CH)], dst_v)

            def row_body(r, carry2):
                for g in range(128 // L):
                    sidx = src_v[r, pl.ds(g * L, L)]
                    didx = dst_v[r, pl.ds(g * L, L)]
                    vals = plsc.load_gather(u_v, [sidx])
                    plsc.addupdate_scatter(acc_v, [didx], vals)
                return carry2

            lax.fori_loop(0, CH, row_body, None)
            return carry

        lax.fori_loop(0, KE // CH, chunk_body, None)

        pltpu.sync_copy(acc_v, out_hbm.at[c, s])

    return k(uT, src2, dst2)


def _tc1(degp, xp, W1):
    def body(degp_ref, x_ref, w_ref, dinv_ref, u_ref):
        deg = degp_ref[:, 0:1] + degp_ref[:, 1:2] + 1.0
        dinv = lax.rsqrt(deg)
        dinv_ref[...] = dinv
        u_ref[...] = jnp.dot(x_ref[...], w_ref[...],
                             preferred_element_type=jnp.float32) * dinv

    return pl.pallas_call(
        body,
        grid=(NP // BM,),
        in_specs=[
            pl.BlockSpec((BM, NC), lambda i: (i, 0)),
            pl.BlockSpec((BM, D_IN), lambda i: (i, 0)),
            pl.BlockSpec((D_IN, F), lambda i: (0, 0)),
        ],
        out_specs=[
            pl.BlockSpec((BM, 1), lambda i: (i, 0)),
            pl.BlockSpec((BM, F), lambda i: (i, 0)),
        ],
        out_shape=[
            jax.ShapeDtypeStruct((NP, 1), jnp.float32),
            jax.ShapeDtypeStruct((NP, F), jnp.float32),
        ],
    )(degp, xp, W1)


def _tc2(u1, p1, dinv, b1, W2p):
    def body(u_ref, p_ref, dinv_ref, b_ref, w_ref, u2_ref):
        tot = u_ref[...] + p_ref[0] + p_ref[1]
        h = jnp.maximum(tot * dinv_ref[...] + b_ref[...], 0.0)
        u2_ref[...] = jnp.dot(h, w_ref[...],
                              preferred_element_type=jnp.float32) * dinv_ref[...]

    return pl.pallas_call(
        body,
        grid=(NP // BM,),
        in_specs=[
            pl.BlockSpec((BM, F), lambda i: (i, 0)),
            pl.BlockSpec((NC, BM, F), lambda i: (0, i, 0)),
            pl.BlockSpec((BM, 1), lambda i: (i, 0)),
            pl.BlockSpec((1, F), lambda i: (0, 0)),
            pl.BlockSpec((F, F), lambda i: (0, 0)),
        ],
        out_specs=pl.BlockSpec((BM, F), lambda i: (i, 0)),
        out_shape=jax.ShapeDtypeStruct((NP, F), jnp.float32),
    )(u1, p1, dinv, b1, W2p)


def _tc3(u2, p2, dinv, b2):
    def body(u_ref, p_ref, dinv_ref, b_ref, o_ref):
        tot = (u_ref[...] + p_ref[0] + p_ref[1]) * dinv_ref[...]
        z = tot[:, :NCLS] + b_ref[...]
        m = jnp.max(z, axis=1, keepdims=True)
        lse = jnp.log(jnp.sum(jnp.exp(z - m), axis=1, keepdims=True)) + m
        o_ref[...] = z - lse

    return pl.pallas_call(
        body,
        grid=(NP // BM,),
        in_specs=[
            pl.BlockSpec((BM, F), lambda i: (i, 0)),
            pl.BlockSpec((NC, BM, F), lambda i: (0, i, 0)),
            pl.BlockSpec((BM, 1), lambda i: (i, 0)),
            pl.BlockSpec((1, NCLS), lambda i: (0, 0)),
        ],
        out_specs=pl.BlockSpec((BM, NCLS), lambda i: (i, 0)),
        out_shape=jax.ShapeDtypeStruct((NP, NCLS), jnp.float32),
    )(u2, p2, dinv, b2)


def kernel(x, edge_index, W1, b1, W2, b2):
    src = edge_index[0]
    dst = edge_index[1]
    pad = jnp.full((EP - E,), N, dtype=jnp.int32)
    dst3 = jnp.concatenate([dst, pad]).reshape(NW, KPW, 128)
    src2 = src.reshape(NC, KE, 128)
    dst2 = dst.reshape(NC, KE, 128)

    xp = jnp.pad(x, ((0, NP - N), (0, 0)))
    W2p = jnp.pad(W2, ((0, 0), (0, F - NCLS)))

    degp = _degree(dst3)
    dinv, u1 = _tc1(degp.T, xp, W1)
    p1 = _aggregate(u1.T, src2, dst2)
    u2 = _tc2(u1, jnp.swapaxes(p1, 1, 2), dinv, b1.reshape(1, F), W2p)
    p2 = _aggregate(u2.T, src2, dst2)
    out = _tc3(u2, jnp.swapaxes(p2, 1, 2), dinv, b2.reshape(1, NCLS))
    return out[:N]


# speedup vs baseline: 1.8157x; 1.8157x over previous
import functools

import jax
import jax.numpy as jnp
from jax import lax
from jax.experimental import pallas as pl
from jax.experimental.pallas import tpu as pltpu
from jax.experimental.pallas import tpu_sc as plsc

N = 10000
NP = 10240
E = 160000
D_IN = 256
F = 16
NCLS = 7

NC = 2
NS = 16
L = 16
NW = NC * NS
KPW = 40
EP = NW * KPW * 128
RPT = NP // NS
BM = 1024


def _sc_mesh():
    return plsc.VectorSubcoreMesh(core_axis_name="c", subcore_axis_name="s")


_SC_PARAMS = pltpu.CompilerParams(needs_layout_passes=False,
                                  use_tc_tiling_on_sc=False)


def _degree(dst3):
    @functools.partial(
        pl.kernel,
        mesh=_sc_mesh(),
        compiler_params=_SC_PARAMS,
        out_type=jax.ShapeDtypeStruct((NC, NP), jnp.float32),
        scratch_types=[
            pltpu.VMEM((KPW, 128), jnp.int32),
            pltpu.VMEM((NP,), jnp.float32),
            pltpu.VMEM_SHARED((NS, NP), jnp.float32),
            pltpu.VMEM((NS, RPT), jnp.float32),
            pltpu.VMEM((RPT,), jnp.float32),
        ],
    )
    def k(dst_hbm, out_hbm, dst_v, deg_v, deg_sh, red_v, sum_v):
        c = lax.axis_index("c")
        s = lax.axis_index("s")
        wid = s * NC + c
        pltpu.sync_copy(dst_hbm.at[wid], dst_v)

        z16 = jnp.zeros((L,), jnp.float32)

        def zero_body(i, carry):
            deg_v[pl.ds(i * L, L)] = z16
            return carry

        lax.fori_loop(0, NP // L, zero_body, None)

        ones16 = jnp.ones((L,), jnp.float32)

        def acc_body(j, carry):
            for l in range(128 // L):
                idx = dst_v[j, pl.ds(l * L, L)]
                plsc.addupdate_scatter(deg_v, [idx], ones16)
            return carry

        lax.fori_loop(0, KPW, acc_body, None)

        pltpu.sync_copy(deg_v, deg_sh.at[s])
        plsc.subcore_barrier()

        for r in range(NS):
            pltpu.sync_copy(deg_sh.at[r, pl.ds(s * RPT, RPT)], red_v.at[r])

        def red_body(t, carry):
            acc = red_v[0, pl.ds(t * L, L)]
            for r in range(1, NS):
                acc = acc + red_v[r, pl.ds(t * L, L)]
            sum_v[pl.ds(t * L, L)] = acc
            return carry

        lax.fori_loop(0, RPT // L, red_body, None)
        pltpu.sync_copy(sum_v, out_hbm.at[c, pl.ds(s * RPT, RPT)])

    return k(dst3)


KE16 = E // NC // L
CHG = 1000
NCH = KE16 // CHG


def _aggregate(uT, src2, dst2):
    @functools.partial(
        pl.kernel,
        mesh=_sc_mesh(),
        compiler_params=_SC_PARAMS,
        out_type=jax.ShapeDtypeStruct((NC, F, NP), jnp.float32),
        scratch_types=[
            pltpu.VMEM((2, CHG, L), jnp.int32),
            pltpu.VMEM((2, CHG, L), jnp.int32),
            pltpu.VMEM((NP,), jnp.float32),
            pltpu.VMEM((NP,), jnp.float32),
            pltpu.SemaphoreType.DMA,
            pltpu.SemaphoreType.DMA,
            pltpu.SemaphoreType.DMA,
            pltpu.SemaphoreType.DMA,
        ],
    )
    def k(u_hbm, src_hbm, dst_hbm, out_hbm, src_v, dst_v, u_v, acc_v,
          sem_s0, sem_s1, sem_d0, sem_d1):
        c = lax.axis_index("c")
        s = lax.axis_index("s")
        pltpu.sync_copy(u_hbm.at[s], u_v)

        z16 = jnp.zeros((L,), jnp.float32)

        @plsc.parallel_loop(0, NP // L, unroll=4)
        def _(i):
            acc_v[pl.ds(i * L, L)] = z16

        ssems = (sem_s0, sem_s1)
        dsems = (sem_d0, sem_d1)

        def start(t, slot):
            pltpu.async_copy(src_hbm.at[c, pl.ds(t * CHG, CHG)],
                             src_v.at[slot], ssems[slot])
            pltpu.async_copy(dst_hbm.at[c, pl.ds(t * CHG, CHG)],
                             dst_v.at[slot], dsems[slot])

        def wait(t, slot):
            pltpu.make_async_copy(src_hbm.at[c, pl.ds(t * CHG, CHG)],
                                  src_v.at[slot], ssems[slot]).wait()
            pltpu.make_async_copy(dst_hbm.at[c, pl.ds(t * CHG, CHG)],
                                  dst_v.at[slot], dsems[slot]).wait()

        start(0, 0)
        for t in range(NCH):
            slot = t % 2
            if t + 1 < NCH:
                start(t + 1, (t + 1) % 2)
            wait(t, slot)

            @plsc.parallel_loop(0, CHG, unroll=4)
            def _(gf):
                sidx = src_v[slot, gf, :]
                didx = dst_v[slot, gf, :]
                vals = plsc.load_gather(u_v, [sidx])
                plsc.addupdate_scatter(acc_v, [didx], vals)

        pltpu.sync_copy(acc_v, out_hbm.at[c, s])

    return k(uT, src2, dst2)


def _tc1(degp, xp, W1):
    def body(degp_ref, x_ref, w_ref, dinv_ref, u_ref):
        deg = degp_ref[:, 0:1] + degp_ref[:, 1:2] + 1.0
        dinv = lax.rsqrt(deg)
        dinv_ref[...] = dinv
        u_ref[...] = jnp.dot(x_ref[...], w_ref[...],
                             preferred_element_type=jnp.float32) * dinv

    return pl.pallas_call(
        body,
        grid=(NP // BM,),
        in_specs=[
            pl.BlockSpec((BM, NC), lambda i: (i, 0)),
            pl.BlockSpec((BM, D_IN), lambda i: (i, 0)),
            pl.BlockSpec((D_IN, F), lambda i: (0, 0)),
        ],
        out_specs=[
            pl.BlockSpec((BM, 1), lambda i: (i, 0)),
            pl.BlockSpec((BM, F), lambda i: (i, 0)),
        ],
        out_shape=[
            jax.ShapeDtypeStruct((NP, 1), jnp.float32),
            jax.ShapeDtypeStruct((NP, F), jnp.float32),
        ],
    )(degp, xp, W1)


def _tc2(u1, p1, dinv, b1, W2p):
    def body(u_ref, p_ref, dinv_ref, b_ref, w_ref, u2_ref):
        tot = u_ref[...] + p_ref[0] + p_ref[1]
        h = jnp.maximum(tot * dinv_ref[...] + b_ref[...], 0.0)
        u2_ref[...] = jnp.dot(h, w_ref[...],
                              preferred_element_type=jnp.float32) * dinv_ref[...]

    return pl.pallas_call(
        body,
        grid=(NP // BM,),
        in_specs=[
            pl.BlockSpec((BM, F), lambda i: (i, 0)),
            pl.BlockSpec((NC, BM, F), lambda i: (0, i, 0)),
            pl.BlockSpec((BM, 1), lambda i: (i, 0)),
            pl.BlockSpec((1, F), lambda i: (0, 0)),
            pl.BlockSpec((F, F), lambda i: (0, 0)),
        ],
        out_specs=pl.BlockSpec((BM, F), lambda i: (i, 0)),
        out_shape=jax.ShapeDtypeStruct((NP, F), jnp.float32),
    )(u1, p1, dinv, b1, W2p)


def _tc3(u2, p2, dinv, b2):
    def body(u_ref, p_ref, dinv_ref, b_ref, o_ref):
        tot = (u_ref[...] + p_ref[0] + p_ref[1]) * dinv_ref[...]
        z = tot[:, :NCLS] + b_ref[...]
        m = jnp.max(z, axis=1, keepdims=True)
        lse = jnp.log(jnp.sum(jnp.exp(z - m), axis=1, keepdims=True)) + m
        o_ref[...] = z - lse

    return pl.pallas_call(
        body,
        grid=(NP // BM,),
        in_specs=[
            pl.BlockSpec((BM, F), lambda i: (i, 0)),
            pl.BlockSpec((NC, BM, F), lambda i: (0, i, 0)),
            pl.BlockSpec((BM, 1), lambda i: (i, 0)),
            pl.BlockSpec((1, NCLS), lambda i: (0, 0)),
        ],
        out_specs=pl.BlockSpec((BM, NCLS), lambda i: (i, 0)),
        out_shape=jax.ShapeDtypeStruct((NP, NCLS), jnp.float32),
    )(u2, p2, dinv, b2)


def kernel(x, edge_index, W1, b1, W2, b2):
    src = edge_index[0]
    dst = edge_index[1]
    pad = jnp.full((EP - E,), N, dtype=jnp.int32)
    dst3 = jnp.concatenate([dst, pad]).reshape(NW, KPW, 128)
    src2 = src.reshape(NC, KE16, L)
    dst2 = dst.reshape(NC, KE16, L)

    xp = jnp.pad(x, ((0, NP - N), (0, 0)))
    W2p = jnp.pad(W2, ((0, 0), (0, F - NCLS)))

    degp = _degree(dst3)
    dinv, u1 = _tc1(degp.T, xp, W1)
    p1 = _aggregate(u1.T, src2, dst2)
    u2 = _tc2(u1, jnp.swapaxes(p1, 1, 2), dinv, b1.reshape(1, F), W2p)
    p2 = _aggregate(u2.T, src2, dst2)
    out = _tc3(u2, jnp.swapaxes(p2, 1, 2), dinv, b2.reshape(1, NCLS))
    return out[:N]


# speedup vs baseline: 1.9240x; 1.0596x over previous
import functools

import jax
import jax.numpy as jnp
from jax import lax
from jax.experimental import pallas as pl
from jax.experimental.pallas import tpu as pltpu
from jax.experimental.pallas import tpu_sc as plsc

N = 10000
NP = 10240
E = 160000
D_IN = 256
F = 16
NCLS = 7

NC = 2
NS = 16
L = 16
NW = NC * NS
KPW = 40
EP = NW * KPW * 128
RPT = NP // NS
BM = 1024


def _sc_mesh():
    return plsc.VectorSubcoreMesh(core_axis_name="c", subcore_axis_name="s")


_SC_PARAMS = pltpu.CompilerParams(needs_layout_passes=False,
                                  use_tc_tiling_on_sc=False)


def _degree(dst3):
    @functools.partial(
        pl.kernel,
        mesh=_sc_mesh(),
        compiler_params=_SC_PARAMS,
        out_type=jax.ShapeDtypeStruct((NC, NP), jnp.float32),
        scratch_types=[
            pltpu.VMEM((KPW * 8, L), jnp.int32),
            pltpu.VMEM((NP,), jnp.float32),
            pltpu.VMEM_SHARED((NS, NP), jnp.float32),
            pltpu.VMEM((NS, RPT), jnp.float32),
            pltpu.VMEM((RPT,), jnp.float32),
            pltpu.SemaphoreType.DMA,
        ],
    )
    def k(dst_hbm, out_hbm, dst_v, deg_v, deg_sh, red_v, sum_v, sem):
        c = lax.axis_index("c")
        s = lax.axis_index("s")
        wid = s * NC + c
        pltpu.async_copy(dst_hbm.at[wid], dst_v, sem)

        z16 = jnp.zeros((L,), jnp.float32)

        @plsc.parallel_loop(0, NP // L, unroll=4)
        def _(i):
            deg_v[pl.ds(i * L, L)] = z16

        pltpu.make_async_copy(dst_hbm.at[wid], dst_v, sem).wait()
        ones16 = jnp.ones((L,), jnp.float32)

        @plsc.parallel_loop(0, KPW * 8, unroll=4)
        def _(g):
            plsc.addupdate_scatter(deg_v, [dst_v[g, :]], ones16)

        pltpu.sync_copy(deg_v, deg_sh.at[s])
        plsc.subcore_barrier()

        for r in range(NS):
            pltpu.sync_copy(deg_sh.at[r, pl.ds(s * RPT, RPT)], red_v.at[r])

        @plsc.parallel_loop(0, RPT // L, unroll=2)
        def _(t):
            acc = red_v[0, pl.ds(t * L, L)]
            for r in range(1, NS):
                acc = acc + red_v[r, pl.ds(t * L, L)]
            sum_v[pl.ds(t * L, L)] = acc

        pltpu.sync_copy(sum_v, out_hbm.at[c, pl.ds(s * RPT, RPT)])

    return k(dst3)


KE16 = E // NC // L
CHG = 1000
NCH = KE16 // CHG


def _aggregate(uT, src2, dst2):
    @functools.partial(
        pl.kernel,
        mesh=_sc_mesh(),
        compiler_params=_SC_PARAMS,
        out_type=jax.ShapeDtypeStruct((NC, F, NP), jnp.float32),
        scratch_types=[
            pltpu.VMEM((2, CHG, L), jnp.int32),
            pltpu.VMEM((2, CHG, L), jnp.int32),
            pltpu.VMEM((NP,), jnp.float32),
            pltpu.VMEM((NP,), jnp.float32),
            pltpu.SemaphoreType.DMA,
            pltpu.SemaphoreType.DMA,
            pltpu.SemaphoreType.DMA,
            pltpu.SemaphoreType.DMA,
        ],
    )
    def k(u_hbm, src_hbm, dst_hbm, out_hbm, src_v, dst_v, u_v, acc_v,
          sem_s0, sem_s1, sem_d0, sem_d1):
        c = lax.axis_index("c")
        s = lax.axis_index("s")

        ssems = (sem_s0, sem_s1)
        dsems = (sem_d0, sem_d1)

        def start(t, slot):
            pltpu.async_copy(src_hbm.at[c, pl.ds(t * CHG, CHG)],
                             src_v.at[slot], ssems[slot])
            pltpu.async_copy(dst_hbm.at[c, pl.ds(t * CHG, CHG)],
                             dst_v.at[slot], dsems[slot])

        def wait(t, slot):
            pltpu.make_async_copy(src_hbm.at[c, pl.ds(t * CHG, CHG)],
                                  src_v.at[slot], ssems[slot]).wait()
            pltpu.make_async_copy(dst_hbm.at[c, pl.ds(t * CHG, CHG)],
                                  dst_v.at[slot], dsems[slot]).wait()

        start(0, 0)
        pltpu.sync_copy(u_hbm.at[s], u_v)

        z16 = jnp.zeros((L,), jnp.float32)

        @plsc.parallel_loop(0, NP // L, unroll=4)
        def _(i):
            acc_v[pl.ds(i * L, L)] = z16

        for t in range(NCH):
            slot = t % 2
            if t + 1 < NCH:
                start(t + 1, (t + 1) % 2)
            wait(t, slot)

            @plsc.parallel_loop(0, CHG, unroll=4)
            def _(gf):
                sidx = src_v[slot, gf, :]
                didx = dst_v[slot, gf, :]
                vals = plsc.load_gather(u_v, [sidx])
                plsc.addupdate_scatter(acc_v, [didx], vals)

        pltpu.sync_copy(acc_v, out_hbm.at[c, s])

    return k(uT, src2, dst2)


def _tc1(degp, xp, W1):
    def body(degp_ref, x_ref, w_ref, dinv_ref, u_ref):
        deg = degp_ref[:, 0:1] + degp_ref[:, 1:2] + 1.0
        dinv = lax.rsqrt(deg)
        dinv_ref[...] = dinv
        u_ref[...] = jnp.dot(x_ref[...], w_ref[...],
                             preferred_element_type=jnp.float32) * dinv

    return pl.pallas_call(
        body,
        grid=(NP // BM,),
        in_specs=[
            pl.BlockSpec((BM, NC), lambda i: (i, 0)),
            pl.BlockSpec((BM, D_IN), lambda i: (i, 0)),
            pl.BlockSpec((D_IN, F), lambda i: (0, 0)),
        ],
        out_specs=[
            pl.BlockSpec((BM, 1), lambda i: (i, 0)),
            pl.BlockSpec((BM, F), lambda i: (i, 0)),
        ],
        out_shape=[
            jax.ShapeDtypeStruct((NP, 1), jnp.float32),
            jax.ShapeDtypeStruct((NP, F), jnp.float32),
        ],
    )(degp, xp, W1)


def _tc2(u1, p1, dinv, b1, W2p):
    def body(u_ref, p_ref, dinv_ref, b_ref, w_ref, u2_ref):
        tot = u_ref[...] + p_ref[0] + p_ref[1]
        h = jnp.maximum(tot * dinv_ref[...] + b_ref[...], 0.0)
        u2_ref[...] = jnp.dot(h, w_ref[...],
                              preferred_element_type=jnp.float32) * dinv_ref[...]

    return pl.pallas_call(
        body,
        grid=(NP // BM,),
        in_specs=[
            pl.BlockSpec((BM, F), lambda i: (i, 0)),
            pl.BlockSpec((NC, BM, F), lambda i: (0, i, 0)),
            pl.BlockSpec((BM, 1), lambda i: (i, 0)),
            pl.BlockSpec((1, F), lambda i: (0, 0)),
            pl.BlockSpec((F, F), lambda i: (0, 0)),
        ],
        out_specs=pl.BlockSpec((BM, F), lambda i: (i, 0)),
        out_shape=jax.ShapeDtypeStruct((NP, F), jnp.float32),
    )(u1, p1, dinv, b1, W2p)


def _tc3(u2, p2, dinv, b2):
    def body(u_ref, p_ref, dinv_ref, b_ref, o_ref):
        tot = (u_ref[...] + p_ref[0] + p_ref[1]) * dinv_ref[...]
        z = tot[:, :NCLS] + b_ref[...]
        m = jnp.max(z, axis=1, keepdims=True)
        lse = jnp.log(jnp.sum(jnp.exp(z - m), axis=1, keepdims=True)) + m
        o_ref[...] = z - lse

    return pl.pallas_call(
        body,
        grid=(NP // BM,),
        in_specs=[
            pl.BlockSpec((BM, F), lambda i: (i, 0)),
            pl.BlockSpec((NC, BM, F), lambda i: (0, i, 0)),
            pl.BlockSpec((BM, 1), lambda i: (i, 0)),
            pl.BlockSpec((1, NCLS), lambda i: (0, 0)),
        ],
        out_specs=pl.BlockSpec((BM, NCLS), lambda i: (i, 0)),
        out_shape=jax.ShapeDtypeStruct((NP, NCLS), jnp.float32),
    )(u2, p2, dinv, b2)


def kernel(x, edge_index, W1, b1, W2, b2):
    src = edge_index[0]
    dst = edge_index[1]
    pad = jnp.full((EP - E,), N, dtype=jnp.int32)
    dst3 = jnp.concatenate([dst, pad]).reshape(NW, KPW * 8, L)
    src2 = src.reshape(NC, KE16, L)
    dst2 = dst.reshape(NC, KE16, L)

    xp = jnp.pad(x, ((0, NP - N), (0, 0)))
    W2p = jnp.pad(W2, ((0, 0), (0, F - NCLS)))

    degp = _degree(dst3)
    dinv, u1 = _tc1(degp.T, xp, W1)
    p1 = _aggregate(u1.T, src2, dst2)
    u2 = _tc2(u1, jnp.swapaxes(p1, 1, 2), dinv, b1.reshape(1, F), W2p)
    p2 = _aggregate(u2.T, src2, dst2)
    out = _tc3(u2, jnp.swapaxes(p2, 1, 2), dinv, b2.reshape(1, NCLS))
    return out[:N]


# speedup vs baseline: 2.5119x; 1.3056x over previous
import functools

import jax
import jax.numpy as jnp
from jax import lax
from jax.experimental import pallas as pl
from jax.experimental.pallas import tpu as pltpu
from jax.experimental.pallas import tpu_sc as plsc

N = 10000
NP = 10240
E = 160000
D_IN = 256
F = 16
NCLS = 7

NC = 2
NS = 16
L = 16
NW = NC * NS
KPW = 40
EP = NW * KPW * 128
RPT = NP // NS
BM = 1024


def _sc_mesh():
    return plsc.VectorSubcoreMesh(core_axis_name="c", subcore_axis_name="s")


_SC_PARAMS = pltpu.CompilerParams(needs_layout_passes=False,
                                  use_tc_tiling_on_sc=False)


def _degree(dst3):
    @functools.partial(
        pl.kernel,
        mesh=_sc_mesh(),
        compiler_params=_SC_PARAMS,
        out_type=jax.ShapeDtypeStruct((NC, NP), jnp.float32),
        scratch_types=[
            pltpu.VMEM((KPW * 8, L), jnp.int32),
            pltpu.VMEM((NP,), jnp.float32),
            pltpu.VMEM_SHARED((NS, NP), jnp.float32),
            pltpu.VMEM((NS, RPT), jnp.float32),
            pltpu.VMEM((RPT,), jnp.float32),
            pltpu.SemaphoreType.DMA,
        ],
    )
    def k(dst_hbm, out_hbm, dst_v, deg_v, deg_sh, red_v, sum_v, sem):
        c = lax.axis_index("c")
        s = lax.axis_index("s")
        wid = s * NC + c
        pltpu.async_copy(dst_hbm.at[wid], dst_v, sem)

        z16 = jnp.zeros((L,), jnp.float32)

        @plsc.parallel_loop(0, NP // L, unroll=4)
        def _(i):
            deg_v[pl.ds(i * L, L)] = z16

        pltpu.make_async_copy(dst_hbm.at[wid], dst_v, sem).wait()
        ones16 = jnp.ones((L,), jnp.float32)

        @plsc.parallel_loop(0, KPW * 8, unroll=4)
        def _(g):
            plsc.addupdate_scatter(deg_v, [dst_v[g, :]], ones16)

        pltpu.sync_copy(deg_v, deg_sh.at[s])
        plsc.subcore_barrier()

        for r in range(NS):
            pltpu.sync_copy(deg_sh.at[r, pl.ds(s * RPT, RPT)], red_v.at[r])

        @plsc.parallel_loop(0, RPT // L, unroll=2)
        def _(t):
            acc = red_v[0, pl.ds(t * L, L)]
            for r in range(1, NS):
                acc = acc + red_v[r, pl.ds(t * L, L)]
            sum_v[pl.ds(t * L, L)] = acc

        pltpu.sync_copy(sum_v, out_hbm.at[c, pl.ds(s * RPT, RPT)])

    return k(dst3)


KE16 = E // NC // L
CHG = 1000
NCH = KE16 // CHG


def _aggregate(uT, src2, dst2):
    @functools.partial(
        pl.kernel,
        mesh=_sc_mesh(),
        compiler_params=_SC_PARAMS,
        out_type=jax.ShapeDtypeStruct((NC, F, NP), jnp.float32),
        scratch_types=[
            pltpu.VMEM((2, CHG, L), jnp.int32),
            pltpu.VMEM((2, CHG, L), jnp.int32),
            pltpu.VMEM((NP,), jnp.float32),
            pltpu.VMEM((NP,), jnp.float32),
            pltpu.SemaphoreType.DMA,
            pltpu.SemaphoreType.DMA,
            pltpu.SemaphoreType.DMA,
            pltpu.SemaphoreType.DMA,
        ],
    )
    def k(u_hbm, src_hbm, dst_hbm, out_hbm, src_v, dst_v, u_v, acc_v,
          sem_s0, sem_s1, sem_d0, sem_d1):
        c = lax.axis_index("c")
        s = lax.axis_index("s")

        ssems = (sem_s0, sem_s1)
        dsems = (sem_d0, sem_d1)

        def start(t, slot):
            pltpu.async_copy(src_hbm.at[c, pl.ds(t * CHG, CHG)],
                             src_v.at[slot], ssems[slot])
            pltpu.async_copy(dst_hbm.at[c, pl.ds(t * CHG, CHG)],
                             dst_v.at[slot], dsems[slot])

        def wait(t, slot):
            pltpu.make_async_copy(src_hbm.at[c, pl.ds(t * CHG, CHG)],
                                  src_v.at[slot], ssems[slot]).wait()
            pltpu.make_async_copy(dst_hbm.at[c, pl.ds(t * CHG, CHG)],
                                  dst_v.at[slot], dsems[slot]).wait()

        start(0, 0)
        pltpu.sync_copy(u_hbm.at[s], u_v)

        z16 = jnp.zeros((L,), jnp.float32)

        @plsc.parallel_loop(0, NP // L, unroll=4)
        def _(i):
            acc_v[pl.ds(i * L, L)] = z16

        for t in range(NCH):
            slot = t % 2
            if t + 1 < NCH:
                start(t + 1, (t + 1) % 2)
            wait(t, slot)

            @plsc.parallel_loop(0, CHG, unroll=4)
            def _(gf):
                sidx = src_v[slot, gf, :]
                didx = dst_v[slot, gf, :]
                vals = plsc.load_gather(u_v, [sidx])
                plsc.addupdate_scatter(acc_v, [didx], vals)

        pltpu.sync_copy(acc_v, out_hbm.at[c, s])

    return k(uT, src2, dst2)


def _tc1(degp, x, W1):
    def body(degp_ref, x_ref, w_ref, dinv_ref, u_ref):
        deg = degp_ref[0:1, :] + degp_ref[1:2, :] + 1.0
        dinv = lax.rsqrt(deg)
        dinv_ref[...] = dinv
        ut = lax.dot_general(w_ref[...], x_ref[...],
                             (((0,), (1,)), ((), ())),
                             preferred_element_type=jnp.float32)
        u_ref[...] = ut * dinv

    return pl.pallas_call(
        body,
        grid=(NP // BM,),
        in_specs=[
            pl.BlockSpec((NC, BM), lambda i: (0, i)),
            pl.BlockSpec((BM, D_IN), lambda i: (i, 0)),
            pl.BlockSpec((D_IN, F), lambda i: (0, 0)),
        ],
        out_specs=[
            pl.BlockSpec((1, BM), lambda i: (0, i)),
            pl.BlockSpec((F, BM), lambda i: (0, i)),
        ],
        out_shape=[
            jax.ShapeDtypeStruct((1, NP), jnp.float32),
            jax.ShapeDtypeStruct((F, NP), jnp.float32),
        ],
    )(degp, x, W1)


def _tc2(u1T, p1, dinvT, b1c, W2p):
    def body(u_ref, p_ref, dinv_ref, b_ref, w_ref, u2_ref):
        tot = u_ref[...] + p_ref[0] + p_ref[1]
        h = jnp.maximum(tot * dinv_ref[...] + b_ref[...], 0.0)
        u2 = lax.dot_general(w_ref[...], h, (((0,), (0,)), ((), ())),
                             preferred_element_type=jnp.float32)
        u2_ref[...] = u2 * dinv_ref[...]

    return pl.pallas_call(
        body,
        grid=(NP // BM,),
        in_specs=[
            pl.BlockSpec((F, BM), lambda i: (0, i)),
            pl.BlockSpec((NC, F, BM), lambda i: (0, 0, i)),
            pl.BlockSpec((1, BM), lambda i: (0, i)),
            pl.BlockSpec((F, 1), lambda i: (0, 0)),
            pl.BlockSpec((F, F), lambda i: (0, 0)),
        ],
        out_specs=pl.BlockSpec((F, BM), lambda i: (0, i)),
        out_shape=jax.ShapeDtypeStruct((F, NP), jnp.float32),
    )(u1T, p1, dinvT, b1c, W2p)


def _tc3(u2T, p2, dinvT, b2c):
    def body(u_ref, p_ref, dinv_ref, b_ref, o_ref):
        tot = (u_ref[...] + p_ref[0] + p_ref[1]) * dinv_ref[...]
        z = tot[:NCLS, :] + b_ref[...]
        m = jnp.max(z, axis=0, keepdims=True)
        lse = jnp.log(jnp.sum(jnp.exp(z - m), axis=0, keepdims=True)) + m
        o_ref[...] = z - lse

    return pl.pallas_call(
        body,
        grid=(NP // BM,),
        in_specs=[
            pl.BlockSpec((F, BM), lambda i: (0, i)),
            pl.BlockSpec((NC, F, BM), lambda i: (0, 0, i)),
            pl.BlockSpec((1, BM), lambda i: (0, i)),
            pl.BlockSpec((NCLS, 1), lambda i: (0, 0)),
        ],
        out_specs=pl.BlockSpec((NCLS, BM), lambda i: (0, i)),
        out_shape=jax.ShapeDtypeStruct((NCLS, NP), jnp.float32),
    )(u2T, p2, dinvT, b2c)


def kernel(x, edge_index, W1, b1, W2, b2):
    src = edge_index[0]
    dst = edge_index[1]
    pad = jnp.full((EP - E,), N, dtype=jnp.int32)
    dst3 = jnp.concatenate([dst, pad]).reshape(NW, KPW * 8, L)
    src2 = src.reshape(NC, KE16, L)
    dst2 = dst.reshape(NC, KE16, L)

    W2p = jnp.pad(W2, ((0, 0), (0, F - NCLS)))

    degp = _degree(dst3)
    dinvT, u1T = _tc1(degp, x, W1)
    p1 = _aggregate(u1T, src2, dst2)
    u2T = _tc2(u1T, p1, dinvT, b1.reshape(F, 1), W2p)
    p2 = _aggregate(u2T, src2, dst2)
    outT = _tc3(u2T, p2, dinvT, b2.reshape(NCLS, 1))
    return outT[:, :N].T


# speedup vs baseline: 2.7208x; 1.0832x over previous
import functools

import jax
import jax.numpy as jnp
from jax import lax
from jax.experimental import pallas as pl
from jax.experimental.pallas import tpu as pltpu
from jax.experimental.pallas import tpu_sc as plsc

N = 10000
NP = 10240
E = 160000
D_IN = 256
F = 16
NCLS = 7

NC = 2
NS = 16
L = 16
NW = NC * NS
KPW = 40
EP = NW * KPW * 128
RPT = NP // NS
BM = 1024


def _sc_mesh():
    return plsc.VectorSubcoreMesh(core_axis_name="c", subcore_axis_name="s",
                                  num_cores=NC, num_subcores=NS)


_SC_PARAMS = pltpu.CompilerParams(needs_layout_passes=False,
                                  use_tc_tiling_on_sc=False)


GPT = (E // L) // NW
REM = (E // L) - NW * GPT


def _degree(dstF):
    @functools.partial(
        pl.kernel,
        mesh=_sc_mesh(),
        compiler_params=_SC_PARAMS,
        out_type=jax.ShapeDtypeStruct((NC, NP), jnp.float32),
        scratch_types=[
            pltpu.VMEM((GPT + REM, L), jnp.int32),
            pltpu.VMEM((NP,), jnp.float32),
            pltpu.VMEM_SHARED((NS, NP), jnp.float32),
            pltpu.VMEM((NS, RPT), jnp.float32),
            pltpu.VMEM((RPT,), jnp.float32),
            pltpu.SemaphoreType.DMA,
        ],
    )
    def k(dst_hbm, out_hbm, dst_v, deg_v, deg_sh, red_v, sum_v, sem):
        c = lax.axis_index("c")
        s = lax.axis_index("s")
        wid = s * NC + c
        pltpu.async_copy(dst_hbm.at[pl.ds(wid * GPT, GPT + REM)], dst_v, sem)

        z16 = jnp.zeros((L,), jnp.float32)

        @plsc.parallel_loop(0, NP // L, unroll=4)
        def _(i):
            deg_v[pl.ds(i * L, L)] = z16

        pltpu.make_async_copy(dst_hbm.at[pl.ds(wid * GPT, GPT + REM)],
                              dst_v, sem).wait()
        ones16 = jnp.ones((L,), jnp.float32)

        @plsc.parallel_loop(0, GPT, unroll=4)
        def _(g):
            plsc.addupdate_scatter(deg_v, [dst_v[g, :]], ones16)

        @pl.when(wid == NW - 1)
        def _():
            @plsc.parallel_loop(GPT, GPT + REM)
            def _(g):
                plsc.addupdate_scatter(deg_v, [dst_v[g, :]], ones16)

        pltpu.sync_copy(deg_v, deg_sh.at[s])
        plsc.subcore_barrier()

        for r in range(NS):
            pltpu.sync_copy(deg_sh.at[r, pl.ds(s * RPT, RPT)], red_v.at[r])

        @plsc.parallel_loop(0, RPT // L, unroll=2)
        def _(t):
            acc = red_v[0, pl.ds(t * L, L)]
            for r in range(1, NS):
                acc = acc + red_v[r, pl.ds(t * L, L)]
            sum_v[pl.ds(t * L, L)] = acc

        pltpu.sync_copy(sum_v, out_hbm.at[c, pl.ds(s * RPT, RPT)])

    return k(dstF)


KE16 = E // L


def _make_aggregate(FW, CHG):
    G = NS // FW
    SPLIT = NC * G
    KEQ = KE16 // SPLIT
    NCH = KEQ // CHG
    assert KEQ % CHG == 0 and NS % FW == 0 and KE16 % SPLIT == 0

    @functools.partial(
        pl.kernel,
        mesh=_sc_mesh(),
        compiler_params=_SC_PARAMS,
        out_type=jax.ShapeDtypeStruct((SPLIT, FW, NP), jnp.float32),
        scratch_types=[
            pltpu.VMEM((2, CHG, L), jnp.int32),
            pltpu.VMEM((2, CHG, L), jnp.int32),
            pltpu.VMEM((NP,), jnp.float32),
            pltpu.VMEM((NP,), jnp.float32),
            pltpu.SemaphoreType.DMA,
            pltpu.SemaphoreType.DMA,
            pltpu.SemaphoreType.DMA,
            pltpu.SemaphoreType.DMA,
        ],
    )
    def k(u_hbm, src_hbm, dst_hbm, out_hbm, src_v, dst_v, u_v, acc_v,
          sem_s0, sem_s1, sem_d0, sem_d1):
        c = lax.axis_index("c")
        s = lax.axis_index("s")
        col = s % FW
        q = c * G + s // FW

        ssems = (sem_s0, sem_s1)
        dsems = (sem_d0, sem_d1)

        def start(t, slot):
            pltpu.async_copy(src_hbm.at[q, pl.ds(t * CHG, CHG)],
                             src_v.at[slot], ssems[slot])
            pltpu.async_copy(dst_hbm.at[q, pl.ds(t * CHG, CHG)],
                             dst_v.at[slot], dsems[slot])

        def wait(t, slot):
            pltpu.make_async_copy(src_hbm.at[q, pl.ds(t * CHG, CHG)],
                                  src_v.at[slot], ssems[slot]).wait()
            pltpu.make_async_copy(dst_hbm.at[q, pl.ds(t * CHG, CHG)],
                                  dst_v.at[slot], dsems[slot]).wait()

        start(0, 0)
        pltpu.sync_copy(u_hbm.at[col], u_v)

        z16 = jnp.zeros((L,), jnp.float32)

        @plsc.parallel_loop(0, NP // L, unroll=4)
        def _(i):
            acc_v[pl.ds(i * L, L)] = z16

        for t in range(NCH):
            slot = t % 2
            if t + 1 < NCH:
                start(t + 1, (t + 1) % 2)
            wait(t, slot)

            @plsc.parallel_loop(0, CHG, unroll=4)
            def _(gf):
                sidx = src_v[slot, gf, :]
                didx = dst_v[slot, gf, :]
                vals = plsc.load_gather(u_v, [sidx])
                plsc.addupdate_scatter(acc_v, [didx], vals)

        pltpu.sync_copy(acc_v, out_hbm.at[q, col])

    return k


_agg16 = _make_aggregate(F, 1000)
_agg8 = _make_aggregate(8, 500)


def _tc1(degp, x, W1):
    def body(degp_ref, x_ref, w_ref, dinv_ref, u_ref):
        deg = degp_ref[0:1, :] + degp_ref[1:2, :] + 1.0
        dinv = lax.rsqrt(deg)
        dinv_ref[...] = dinv
        ut = lax.dot_general(w_ref[...], x_ref[...],
                             (((0,), (1,)), ((), ())),
                             preferred_element_type=jnp.float32)
        u_ref[...] = ut * dinv

    return pl.pallas_call(
        body,
        grid=(NP // BM,),
        in_specs=[
            pl.BlockSpec((NC, BM), lambda i: (0, i)),
            pl.BlockSpec((BM, D_IN), lambda i: (i, 0)),
            pl.BlockSpec((D_IN, F), lambda i: (0, 0)),
        ],
        out_specs=[
            pl.BlockSpec((1, BM), lambda i: (0, i)),
            pl.BlockSpec((F, BM), lambda i: (0, i)),
        ],
        out_shape=[
            jax.ShapeDtypeStruct((1, NP), jnp.float32),
            jax.ShapeDtypeStruct((F, NP), jnp.float32),
        ],
    )(degp, x, W1)


F8 = 8


def _tc2(u1T, p1, dinvT, b1c, W2):
    def body(u_ref, p_ref, dinv_ref, b_ref, w_ref, u2_ref):
        tot = u_ref[...] + p_ref[0] + p_ref[1]
        h = jnp.maximum(tot * dinv_ref[...] + b_ref[...], 0.0)
        u2 = lax.dot_general(w_ref[...], h, (((0,), (0,)), ((), ())),
                             preferred_element_type=jnp.float32)
        u2_ref[0:NCLS, :] = u2 * dinv_ref[...]
        u2_ref[NCLS:F8, :] = jnp.zeros((F8 - NCLS, BM), jnp.float32)

    return pl.pallas_call(
        body,
        grid=(NP // BM,),
        in_specs=[
            pl.BlockSpec((F, BM), lambda i: (0, i)),
            pl.BlockSpec((NC, F, BM), lambda i: (0, 0, i)),
            pl.BlockSpec((1, BM), lambda i: (0, i)),
            pl.BlockSpec((F, 1), lambda i: (0, 0)),
            pl.BlockSpec((F, NCLS), lambda i: (0, 0)),
        ],
        out_specs=pl.BlockSpec((F8, BM), lambda i: (0, i)),
        out_shape=jax.ShapeDtypeStruct((F8, NP), jnp.float32),
    )(u1T, p1, dinvT, b1c, W2)


def _tc3(u2T, p2, dinvT, b2c):
    def body(u_ref, p_ref, dinv_ref, b_ref, o_ref):
        tot = (u_ref[...] + p_ref[0] + p_ref[1] + p_ref[2] + p_ref[3])
        tot = tot * dinv_ref[...]
        z = tot[:NCLS, :] + b_ref[...]
        m = jnp.max(z, axis=0, keepdims=True)
        lse = jnp.log(jnp.sum(jnp.exp(z - m), axis=0, keepdims=True)) + m
        o_ref[...] = (z - lse).T

    return pl.pallas_call(
        body,
        grid=(NP // BM,),
        in_specs=[
            pl.BlockSpec((F8, BM), lambda i: (0, i)),
            pl.BlockSpec((4, F8, BM), lambda i: (0, 0, i)),
            pl.BlockSpec((1, BM), lambda i: (0, i)),
            pl.BlockSpec((NCLS, 1), lambda i: (0, 0)),
        ],
        out_specs=pl.BlockSpec((BM, NCLS), lambda i: (i, 0)),
        out_shape=jax.ShapeDtypeStruct((N, NCLS), jnp.float32),
    )(u2T, p2, dinvT, b2c)


def kernel(x, edge_index, W1, b1, W2, b2):
    src = edge_index[0]
    dst = edge_index[1]
    srcQ1 = src.reshape(NC, KE16 // NC, L)
    dstQ1 = dst.reshape(NC, KE16 // NC, L)
    srcQ2 = src.reshape(4, KE16 // 4, L)
    dstQ2 = dst.reshape(4, KE16 // 4, L)
    dstF = dst.reshape(KE16, L)

    degp = _degree(dstF)
    dinvT, u1T = _tc1(degp, x, W1)
    p1 = _agg16(u1T, srcQ1, dstQ1)
    u2T = _tc2(u1T, p1, dinvT, b1.reshape(F, 1), W2)
    p2 = _agg8(u2T, srcQ2, dstQ2)
    return _tc3(u2T, p2, dinvT, b2.reshape(NCLS, 1))
